# baseline (speedup 1.0000x reference)
import functools

import jax
import jax.numpy as jnp
from jax import lax
from jax.experimental import pallas as pl
from jax.experimental.pallas import tpu as pltpu
from jax.experimental.pallas import tpu_sc as plsc

NUM_CORES = 2
NUM_SUBCORES = 16
NUM_WORKERS = NUM_CORES * NUM_SUBCORES
LANES = 16
NBUF = 4


def _make_kernel(n_rows, d, chunk):
    assert n_rows % (NUM_WORKERS * chunk) == 0
    rows_per_worker = n_rows // NUM_WORKERS
    n_chunks = rows_per_worker // chunk
    assert n_chunks >= NBUF
    d_vregs = d // LANES

    mesh = plsc.VectorSubcoreMesh(
        core_axis_name="c", subcore_axis_name="s")

    scratch = [
        pltpu.VMEM((rows_per_worker,), jnp.int32),
        pltpu.VMEM((rows_per_worker,), jnp.int32),
    ]
    for _ in range(NBUF):
        scratch.append(pltpu.VMEM((chunk, d), jnp.float32))
        scratch.append(pltpu.VMEM((chunk, d // 2), jnp.int32))
        scratch.append(pltpu.SemaphoreType.DMA)
        scratch.append(pltpu.SemaphoreType.DMA)

    @functools.partial(
        pl.kernel,
        mesh=mesh,
        out_type=jax.ShapeDtypeStruct((n_rows, d), jnp.float32),
        compiler_params=pltpu.CompilerParams(needs_layout_passes=False),
        scratch_types=scratch,
    )
    def kern(tok_ids_hbm, pos_ids_hbm, tok_tab_hbm, pos_tab_hbm, out_hbm,
             tid_all, pid_all, *bufs):
        rings = [tuple(bufs[4 * k:4 * k + 4]) for k in range(NBUF)]
        wid = lax.axis_index("s") * NUM_CORES + lax.axis_index("c")
        base = wid * rows_per_worker

        pltpu.sync_copy(tok_ids_hbm.at[pl.ds(base, rows_per_worker)], tid_all)
        pltpu.sync_copy(pos_ids_hbm.at[pl.ds(base, rows_per_worker)], pid_all)

        def gathers(j, ring):
            tbuf, pbuf, g, _ = ring
            sl = pl.ds(j * chunk, chunk)
            pltpu.async_copy(tok_tab_hbm.at[tid_all.at[sl]], tbuf, g)
            pltpu.async_copy(pos_tab_hbm.at[pid_all.at[sl]], pbuf, g)

        def wait_gathers(ring):
            tbuf, pbuf, g, _ = ring
            pltpu.make_async_copy(
                tok_tab_hbm.at[pl.ds(0, chunk)], tbuf, g).wait()
            pltpu.make_async_copy(
                pos_tab_hbm.at[pl.ds(0, chunk)], pbuf, g).wait()

        def wait_writeback(ring):
            tbuf, _, _, o = ring
            pltpu.make_async_copy(
                tbuf, out_hbm.at[pl.ds(0, chunk)], o).wait()

        for k in range(NBUF - 1):
            gathers(k, rings[k])

        def process(j, k):
            ring = rings[k]
            nxt = rings[(k + NBUF - 1) % NBUF]
            @pl.when(j + NBUF - 1 < n_chunks)
            def _():
                @pl.when(j >= 1)
                def _():
                    wait_writeback(nxt)
                gathers(j + NBUF - 1, nxt)

            wait_gathers(ring)
            tbuf, pbuf, _, o = ring

            def row_body(r, carry):
                for c in range(d_vregs // 2):
                    pb32 = pbuf[r, pl.ds(c * LANES, LANES)]
                    ab = plsc.bitcast(pb32, jnp.bfloat16)
                    a, b = plsc.unpack(ab, format=plsc.PackFormat.INTERLEAVED)
                    plsc.addupdate(tbuf.at[r, pl.ds(c * 2 * LANES, LANES)], a)
                    plsc.addupdate(
                        tbuf.at[r, pl.ds(c * 2 * LANES + LANES, LANES)], b)
                return carry

            lax.fori_loop(0, chunk, row_body, 0)
            pltpu.async_copy(
                tbuf, out_hbm.at[pl.ds(base + j * chunk, chunk)], o)

        def chunk_body(j, carry):
            for k in range(NBUF):
                @pl.when(j % NBUF == k)
                def _(k=k):
                    process(j, k)

            return carry

        lax.fori_loop(0, n_chunks, chunk_body, 0)
        for k in range(NBUF):
            wait_writeback(rings[k])

    return kern


def kernel(input_ids, position_ids, token_table, position_table):
    b, w = input_ids.shape
    n_pos, d = position_table.shape
    n_rows = b * w
    rows_per_worker = n_rows // NUM_WORKERS
    flat_tok = input_ids.T.reshape(n_rows).astype(jnp.int32)
    flat_pos = position_ids.T.reshape(n_rows).astype(jnp.int32)
    cols = jnp.arange(d, dtype=jnp.int32)
    within = cols % (2 * LANES)
    perm = (cols // (2 * LANES)) * (2 * LANES) + jnp.where(
        within % 2 == 0, within // 2, LANES + within // 2)
    pos_bf16 = position_table.astype(jnp.bfloat16)[:, perm]
    pos_i32 = lax.bitcast_convert_type(
        pos_bf16.reshape(n_pos, d // 2, 2), jnp.int32)
    pos_rep = jnp.tile(pos_i32, (NUM_WORKERS, 1))
    replica = (jnp.arange(n_rows, dtype=jnp.int32) // rows_per_worker) * n_pos
    flat_pos = flat_pos + replica
    kern = _make_kernel(n_rows, d, chunk=16)
    out = kern(flat_tok, flat_pos, token_table, pos_rep)
    return out.reshape(w, b, d).transpose(1, 0, 2)


# speedup vs baseline: 1.3234x; 1.3234x over previous
import functools

import jax
import jax.numpy as jnp
from jax import lax
from jax.experimental import pallas as pl
from jax.experimental.pallas import tpu as pltpu
from jax.experimental.pallas import tpu_sc as plsc

NUM_CORES = 2
NUM_SUBCORES = 16
NUM_WORKERS = NUM_CORES * NUM_SUBCORES
LANES = 16
NBUF = 5


def _make_kernel(n_rows, d, chunk):
    assert n_rows % (NUM_WORKERS * chunk) == 0
    rows_per_worker = n_rows // NUM_WORKERS
    n_chunks = rows_per_worker // chunk
    assert n_chunks >= NBUF
    d_vregs = d // LANES

    mesh = plsc.VectorSubcoreMesh(
        core_axis_name="c", subcore_axis_name="s")

    scratch = [
        pltpu.VMEM((rows_per_worker,), jnp.int32),
        pltpu.VMEM((rows_per_worker,), jnp.int32),
    ]
    for _ in range(NBUF):
        scratch.append(pltpu.VMEM((chunk, d), jnp.float32))
        scratch.append(pltpu.VMEM((chunk, d), jnp.float32))
        scratch.append(pltpu.SemaphoreType.DMA)
        scratch.append(pltpu.SemaphoreType.DMA)

    @functools.partial(
        pl.kernel,
        mesh=mesh,
        out_type=jax.ShapeDtypeStruct((n_rows, d), jnp.float32),
        scratch_types=scratch,
    )
    def kern(tok_ids_hbm, pos_ids_hbm, tok_tab_hbm, pos_tab_hbm, out_hbm,
             tid_all, pid_all, *bufs):
        rings = [tuple(bufs[4 * k:4 * k + 4]) for k in range(NBUF)]
        wid = lax.axis_index("s") * NUM_CORES + lax.axis_index("c")
        base = wid * rows_per_worker

        pltpu.sync_copy(tok_ids_hbm.at[pl.ds(base, rows_per_worker)], tid_all)
        pltpu.sync_copy(pos_ids_hbm.at[pl.ds(base, rows_per_worker)], pid_all)

        def gathers(j, ring):
            tbuf, pbuf, g, _ = ring
            sl = pl.ds(j * chunk, chunk)
            pltpu.async_copy(tok_tab_hbm.at[tid_all.at[sl]], tbuf, g)
            pltpu.async_copy(pos_tab_hbm.at[pid_all.at[sl]], pbuf, g)

        def wait_gathers(ring):
            tbuf, pbuf, g, _ = ring
            pltpu.make_async_copy(
                tok_tab_hbm.at[pl.ds(0, chunk)], tbuf, g).wait()
            pltpu.make_async_copy(
                pos_tab_hbm.at[pl.ds(0, chunk)], pbuf, g).wait()

        def wait_writeback(ring):
            tbuf, _, _, o = ring
            pltpu.make_async_copy(
                tbuf, out_hbm.at[pl.ds(0, chunk)], o).wait()

        for k in range(NBUF - 1):
            gathers(k, rings[k])

        def process(j, k):
            ring = rings[k]
            nxt = rings[(k + NBUF - 1) % NBUF]
            @pl.when(j + NBUF - 1 < n_chunks)
            def _():
                @pl.when(j >= 1)
                def _():
                    wait_writeback(nxt)
                gathers(j + NBUF - 1, nxt)

            wait_gathers(ring)
            tbuf, pbuf, _, o = ring

            def row_body(r, carry):
                for c in range(d_vregs):
                    sl = pl.ds(c * LANES, LANES)
                    plsc.addupdate(tbuf.at[r, sl], pbuf[r, sl])
                return carry

            lax.fori_loop(0, chunk, row_body, 0)
            pltpu.async_copy(
                tbuf, out_hbm.at[pl.ds(base + j * chunk, chunk)], o)

        def chunk_body(j, carry):
            for k in range(NBUF):
                @pl.when(j % NBUF == k)
                def _(k=k):
                    process(j, k)

            return carry

        lax.fori_loop(0, n_chunks, chunk_body, 0)
        for k in range(NBUF):
            wait_writeback(rings[k])

    return kern


def kernel(input_ids, position_ids, token_table, position_table):
    b, w = input_ids.shape
    n_pos, d = position_table.shape
    n_rows = b * w
    rows_per_worker = n_rows // NUM_WORKERS
    flat_tok = input_ids.T.reshape(n_rows).astype(jnp.int32)
    flat_pos = position_ids.T.reshape(n_rows).astype(jnp.int32)
    pos_rep = jnp.tile(position_table, (NUM_WORKERS, 1))
    replica = (jnp.arange(n_rows, dtype=jnp.int32) // rows_per_worker) * n_pos
    flat_pos = flat_pos + replica
    kern = _make_kernel(n_rows, d, chunk=16)
    out = kern(flat_tok, flat_pos, token_table, pos_rep)
    return out.reshape(w, b, d).transpose(1, 0, 2)


# speedup vs baseline: 1.3384x; 1.0113x over previous
import functools

import jax
import jax.numpy as jnp
from jax import lax
from jax.experimental import pallas as pl
from jax.experimental.pallas import tpu as pltpu
from jax.experimental.pallas import tpu_sc as plsc

NUM_CORES = 2
NUM_SUBCORES = 16
NUM_WORKERS = NUM_CORES * NUM_SUBCORES
LANES = 16
NBUF = 2


def _make_kernel(n_rows, d, chunk):
    assert n_rows % (NUM_WORKERS * chunk) == 0
    rows_per_worker = n_rows // NUM_WORKERS
    n_chunks = rows_per_worker // chunk
    assert n_chunks >= NBUF
    d_vregs = d // LANES

    mesh = plsc.VectorSubcoreMesh(
        core_axis_name="c", subcore_axis_name="s")

    scratch = [
        pltpu.VMEM((rows_per_worker,), jnp.int32),
        pltpu.VMEM((rows_per_worker,), jnp.int32),
    ]
    for _ in range(NBUF):
        scratch.append(pltpu.VMEM((chunk, d), jnp.float32))
        scratch.append(pltpu.VMEM((chunk, d), jnp.float32))
        scratch.append(pltpu.SemaphoreType.DMA)
        scratch.append(pltpu.SemaphoreType.DMA)

    @functools.partial(
        pl.kernel,
        mesh=mesh,
        out_type=jax.ShapeDtypeStruct((n_rows, d), jnp.float32),
        scratch_types=scratch,
    )
    def kern(tok_ids_hbm, pos_ids_hbm, tok_tab_hbm, pos_tab_hbm, out_hbm,
             tid_all, pid_all, *bufs):
        rings = [tuple(bufs[4 * k:4 * k + 4]) for k in range(NBUF)]
        wid = lax.axis_index("s") * NUM_CORES + lax.axis_index("c")
        base = wid * rows_per_worker

        pltpu.sync_copy(tok_ids_hbm.at[pl.ds(base, rows_per_worker)], tid_all)
        pltpu.sync_copy(pos_ids_hbm.at[pl.ds(base, rows_per_worker)], pid_all)

        def gathers(j, ring):
            tbuf, pbuf, g, _ = ring
            sl = pl.ds(j * chunk, chunk)
            pltpu.async_copy(tok_tab_hbm.at[tid_all.at[sl]], tbuf, g)
            pltpu.async_copy(pos_tab_hbm.at[pid_all.at[sl]], pbuf, g)

        def wait_gathers(ring):
            tbuf, pbuf, g, _ = ring
            pltpu.make_async_copy(
                tok_tab_hbm.at[pl.ds(0, chunk)], tbuf, g).wait()
            pltpu.make_async_copy(
                pos_tab_hbm.at[pl.ds(0, chunk)], pbuf, g).wait()

        def wait_writeback(ring):
            tbuf, _, _, o = ring
            pltpu.make_async_copy(
                tbuf, out_hbm.at[pl.ds(0, chunk)], o).wait()

        for k in range(NBUF - 1):
            gathers(k, rings[k])

        def process(j, k):
            ring = rings[k]
            nxt = rings[(k + NBUF - 1) % NBUF]
            @pl.when(j + NBUF - 1 < n_chunks)
            def _():
                @pl.when(j >= 1)
                def _():
                    wait_writeback(nxt)
                gathers(j + NBUF - 1, nxt)

            wait_gathers(ring)
            tbuf, pbuf, _, o = ring

            def row_body(r, carry):
                for c in range(d_vregs):
                    sl = pl.ds(c * LANES, LANES)
                    plsc.addupdate(tbuf.at[r, sl], pbuf[r, sl])
                return carry

            lax.fori_loop(0, chunk, row_body, 0)
            pltpu.async_copy(
                tbuf, out_hbm.at[pl.ds(base + j * chunk, chunk)], o)

        def chunk_body(j, carry):
            for k in range(NBUF):
                @pl.when(j % NBUF == k)
                def _(k=k):
                    process(j, k)

            return carry

        lax.fori_loop(0, n_chunks, chunk_body, 0)
        for k in range(NBUF):
            wait_writeback(rings[k])

    return kern


def kernel(input_ids, position_ids, token_table, position_table):
    b, w = input_ids.shape
    n_pos, d = position_table.shape
    n_rows = b * w
    rows_per_worker = n_rows // NUM_WORKERS
    flat_tok = input_ids.T.reshape(n_rows).astype(jnp.int32)
    flat_pos = position_ids.T.reshape(n_rows).astype(jnp.int32)
    pos_rep = jnp.tile(position_table, (NUM_WORKERS, 1))
    replica = (jnp.arange(n_rows, dtype=jnp.int32) // rows_per_worker) * n_pos
    flat_pos = flat_pos + replica
    kern = _make_kernel(n_rows, d, chunk=32)
    out = kern(flat_tok, flat_pos, token_table, pos_rep)
    return out.reshape(w, b, d).transpose(1, 0, 2)
